# baseline (speedup 1.0000x reference)
import functools

import jax
import jax.numpy as jnp
from jax.experimental import pallas as pl
from jax.experimental.pallas import tpu as pltpu

BATCH = 128
VOCAB = 100000
CHUNK = 1024
BLOCK_W = 25600
NUM_BLOCKS = (VOCAB + BLOCK_W - 1) // BLOCK_W
CHUNKS_PER_BLOCK = BLOCK_W // CHUNK
ROWS_PER_GROUP = 8
NUM_GROUPS = BATCH // ROWS_PER_GROUP

_KEY0 = 928981903
_KEY1 = 3453687069
_KS2 = _KEY0 ^ _KEY1 ^ 0x1BD11BDA


def _u32(x):
    return jnp.uint32(x)


def _rotl(x, d):
    return (x << _u32(d)) | (x >> _u32(32 - d))


def _threefry2x32_zero_hi(x1):
    ks = (_KEY0, _KEY1, _KS2)
    rot0 = (13, 15, 26, 6)
    rot1 = (17, 29, 16, 24)
    x0 = x1 + _u32(_KEY0)
    x1 = _rotl(x1, rot0[0]) ^ x0
    first = True
    for rots, ka, kb, inc in (
        (rot0, 1, 2, 1),
        (rot1, 2, 0, 2),
        (rot0, 0, 1, 3),
        (rot1, 1, 2, 4),
        (rot0, 2, 0, 5),
    ):
        for r in rots:
            if first:
                first = False
                continue
            x0 = x0 + x1
            x1 = _rotl(x1, r)
            x1 = x1 ^ x0
        x0 = x0 + _u32(ks[ka])
        x1 = x1 + _u32((ks[kb] + inc) & 0xFFFFFFFF)
    return x0 ^ x1


_TINY = float(jnp.finfo(jnp.float32).tiny)


def _sample_block(logits_ref, out_ref, best_val, best_idx):
    s = pl.program_id(0)
    j = pl.program_id(1)
    cshape = (ROWS_PER_GROUP, CHUNK)

    row_off = jax.lax.broadcasted_iota(jnp.uint32, cshape, 0) * _u32(VOCAB)
    lane = jax.lax.broadcasted_iota(jnp.uint32, cshape, 1)
    base_vec = row_off + lane
    lane_i32 = lane.astype(jnp.int32)

    col_block0 = j * BLOCK_W
    group_scalar = (
        s.astype(jnp.uint32) * _u32(ROWS_PER_GROUP * VOCAB)
        + _u32(_KEY1)
        + col_block0.astype(jnp.uint32)
    )

    @pl.when(j == 0)
    def _init():
        best_val[...] = jnp.full(cshape, -jnp.inf, jnp.float32)
        best_idx[...] = jnp.zeros(cshape, jnp.int32)

    vrun = best_val[...]
    irun = best_idx[...]

    for k in range(CHUNKS_PER_BLOCK):
        c0 = k * CHUNK
        x1 = base_vec + (group_scalar + _u32(c0))
        bits = _threefry2x32_zero_hi(x1)
        fb = (bits >> _u32(9)) | _u32(0x3F800000)
        f = pltpu.bitcast(fb, jnp.float32) - jnp.float32(1.0)
        g = -jnp.log(-jnp.log(jnp.maximum(f, jnp.float32(_TINY))))
        col = lane_i32 + (col_block0 + c0)
        v = jnp.where(col < VOCAB, logits_ref[:, c0 : c0 + CHUNK] + g, -jnp.inf)
        take = v > vrun
        vrun = jnp.maximum(vrun, v)
        irun = jnp.where(take, col, irun)

    best_val[...] = vrun
    best_idx[...] = irun

    @pl.when(j == NUM_BLOCKS - 1)
    def _done():
        gmax = jnp.max(vrun, axis=1, keepdims=True)
        cand = jnp.where(vrun == gmax, irun, jnp.int32(0x7FFFFFFF))
        out_ref[...] = jnp.min(cand, axis=1, keepdims=True)


@functools.partial(jax.jit, static_argnames=())
def kernel(logits):
    out = pl.pallas_call(
        _sample_block,
        grid=(NUM_GROUPS, NUM_BLOCKS),
        in_specs=[
            pl.BlockSpec((ROWS_PER_GROUP, BLOCK_W), lambda s, j: (s, j)),
        ],
        out_specs=pl.BlockSpec((ROWS_PER_GROUP, 1), lambda s, j: (s, 0)),
        out_shape=jax.ShapeDtypeStruct((BATCH, 1), jnp.int32),
        scratch_shapes=[
            pltpu.VMEM((ROWS_PER_GROUP, CHUNK), jnp.float32),
            pltpu.VMEM((ROWS_PER_GROUP, CHUNK), jnp.int32),
        ],
        compiler_params=pltpu.CompilerParams(
            dimension_semantics=("arbitrary", "arbitrary"),
        ),
    )(logits)
    return out.reshape(BATCH)


# speedup vs baseline: 1.0357x; 1.0357x over previous
import functools

import jax
import jax.numpy as jnp
from jax.experimental import pallas as pl
from jax.experimental.pallas import tpu as pltpu

BATCH = 128
VOCAB = 100000
CHUNK = 1024
ROWS_PER_GROUP = 8
NUM_GROUPS = BATCH // ROWS_PER_GROUP

_STARTS = list(range(0, VOCAB - CHUNK + 1, CHUNK))
if _STARTS[-1] + CHUNK < VOCAB:
    _STARTS.append(VOCAB - CHUNK)

_KEY0 = 928981903
_KEY1 = 3453687069
_KS2 = _KEY0 ^ _KEY1 ^ 0x1BD11BDA


def _u32(x):
    return jnp.uint32(x)


def _rotl(x, d):
    return (x << _u32(d)) | (x >> _u32(32 - d))


def _threefry2x32_zero_hi(x1):
    ks = (_KEY0, _KEY1, _KS2)
    rot0 = (13, 15, 26, 6)
    rot1 = (17, 29, 16, 24)
    x0 = x1 + _u32(_KEY0)
    x1 = _rotl(x1, rot0[0]) ^ x0
    first = True
    for rots, ka, kb, inc in (
        (rot0, 1, 2, 1),
        (rot1, 2, 0, 2),
        (rot0, 0, 1, 3),
        (rot1, 1, 2, 4),
        (rot0, 2, 0, 5),
    ):
        for r in rots:
            if first:
                first = False
                continue
            x0 = x0 + x1
            x1 = _rotl(x1, r)
            x1 = x1 ^ x0
        x0 = x0 + _u32(ks[ka])
        x1 = x1 + _u32((ks[kb] + inc) & 0xFFFFFFFF)
    return x0 ^ x1


_TINY = float(jnp.finfo(jnp.float32).tiny)


def _slab_copy(logits_hbm, buf, sem, si, slot):
    return pltpu.make_async_copy(
        logits_hbm.at[pl.ds(si * ROWS_PER_GROUP, ROWS_PER_GROUP), :],
        buf.at[slot],
        sem.at[slot],
    )


def _sample_kernel(logits_hbm, out_ref, buf, sem):
    cshape = (ROWS_PER_GROUP, CHUNK)

    row_off = jax.lax.broadcasted_iota(jnp.uint32, cshape, 0) * _u32(VOCAB)
    lane = jax.lax.broadcasted_iota(jnp.uint32, cshape, 1)
    base_vec = row_off + lane
    lane_i32 = lane.astype(jnp.int32)

    _slab_copy(logits_hbm, buf, sem, 0, 0).start()

    def step(s, carry):
        slot = jax.lax.rem(s, 2)
        nxt = s + 1

        @pl.when(nxt < NUM_GROUPS)
        def _prefetch():
            _slab_copy(logits_hbm, buf, sem, nxt, 1 - slot).start()

        _slab_copy(logits_hbm, buf, sem, s, slot).wait()
        slab = buf.at[slot]

        group_scalar = s.astype(jnp.uint32) * _u32(
            ROWS_PER_GROUP * VOCAB
        ) + _u32(_KEY1)

        vrun = jnp.full(cshape, -jnp.inf, jnp.float32)
        irun = jnp.zeros(cshape, jnp.int32)
        for c0 in _STARTS:
            x1 = base_vec + (group_scalar + _u32(c0))
            bits = _threefry2x32_zero_hi(x1)
            fb = (bits >> _u32(9)) | _u32(0x3F800000)
            f = pltpu.bitcast(fb, jnp.float32) - jnp.float32(1.0)
            g = -jnp.log(-jnp.log(jnp.maximum(f, jnp.float32(_TINY))))
            v = slab[:, c0 : c0 + CHUNK] + g
            col = lane_i32 + c0
            take = v > vrun
            vrun = jnp.maximum(vrun, v)
            irun = jnp.where(take, col, irun)

        gmax = jnp.max(vrun, axis=1, keepdims=True)
        cand = jnp.where(vrun == gmax, irun, jnp.int32(0x7FFFFFFF))
        gidx = jnp.min(cand, axis=1, keepdims=True)
        out_ref[pl.ds(s * ROWS_PER_GROUP, ROWS_PER_GROUP), :] = gidx
        return carry

    jax.lax.fori_loop(0, NUM_GROUPS, step, 0)


@functools.partial(jax.jit, static_argnames=())
def kernel(logits):
    out = pl.pallas_call(
        _sample_kernel,
        in_specs=[pl.BlockSpec(memory_space=pl.ANY)],
        out_specs=pl.BlockSpec(memory_space=pltpu.MemorySpace.VMEM),
        out_shape=jax.ShapeDtypeStruct((BATCH, 1), jnp.int32),
        scratch_shapes=[
            pltpu.VMEM((2, ROWS_PER_GROUP, VOCAB), jnp.float32),
            pltpu.SemaphoreType.DMA((2,)),
        ],
    )(logits)
    return out.reshape(BATCH)
